# Initial kernel scaffold; baseline (speedup 1.0000x reference)
import jax
import jax.numpy as jnp
from jax.experimental import pallas as pl


def kernel(node_type, in_degree, out_degree, node_table, in_degree_table, out_degree_table):
    raise NotImplementedError("write your pallas kernel here")


# speedup vs baseline: 1.8990x; 1.8990x over previous
import functools

import jax
import jax.numpy as jnp
from jax import lax
from jax.experimental import pallas as pl
from jax.experimental.pallas import tpu as pltpu
from jax.experimental.pallas import tpu_sc as plsc

NC = 2
NS = 16
NW = NC * NS
L = 16

EMBED = 768
R_TOTAL = 1024 * 128
ROWS_PER_W = R_TOTAL // NW
CHUNK = 32
N_CHUNKS = ROWS_PER_W // CHUNK
VREGS_PER_ROW = EMBED // L


def _sc_kernel(nt_hbm, in_hbm, ot_hbm, node_tab, in_tab, out_tab, out_hbm,
               idx_n, idx_i, idx_o, buf_n, buf_i, buf_o, sem_i, sem_g):
    wid = lax.axis_index("s") * NC + lax.axis_index("c")
    w_base = wid * ROWS_PER_W

    @pl.loop(0, N_CHUNKS)
    def _chunk(c):
        base = w_base + c * CHUNK
        ci = pltpu.async_copy(nt_hbm.at[pl.ds(base, CHUNK)], idx_n, sem_i)
        cj = pltpu.async_copy(in_hbm.at[pl.ds(base, CHUNK)], idx_i, sem_i)
        ck = pltpu.async_copy(ot_hbm.at[pl.ds(base, CHUNK)], idx_o, sem_i)
        ci.wait()
        cj.wait()
        ck.wait()
        g0 = pltpu.async_copy(node_tab.at[idx_n], buf_n, sem_g)
        g1 = pltpu.async_copy(in_tab.at[idx_i], buf_i, sem_g)
        g2 = pltpu.async_copy(out_tab.at[idx_o], buf_o, sem_g)
        g0.wait()
        g1.wait()
        g2.wait()

        @pl.loop(0, CHUNK)
        def _row(r):
            for k in range(VREGS_PER_ROW):
                sl = pl.ds(k * L, L)
                buf_n[r, sl] = buf_n[r, sl] + buf_i[r, sl] + buf_o[r, sl]

        pltpu.sync_copy(buf_n, out_hbm.at[pl.ds(base, CHUNK)])


@jax.jit
def _run(nt, ind, outd, node_tab, in_tab, out_tab):
    mesh = plsc.VectorSubcoreMesh(
        core_axis_name="c", subcore_axis_name="s", num_cores=NC,
        num_subcores=NS)
    f = pl.kernel(
        _sc_kernel,
        out_type=jax.ShapeDtypeStruct((R_TOTAL, EMBED), jnp.float32),
        mesh=mesh,
        scratch_types=[
            pltpu.VMEM((CHUNK,), jnp.int32),
            pltpu.VMEM((CHUNK,), jnp.int32),
            pltpu.VMEM((CHUNK,), jnp.int32),
            pltpu.VMEM((CHUNK, EMBED), jnp.float32),
            pltpu.VMEM((CHUNK, EMBED), jnp.float32),
            pltpu.VMEM((CHUNK, EMBED), jnp.float32),
            pltpu.SemaphoreType.DMA,
            pltpu.SemaphoreType.DMA,
        ],
    )
    return f(nt, ind, outd, node_tab, in_tab, out_tab)


def kernel(node_type, in_degree, out_degree, node_table, in_degree_table,
           out_degree_table):
    n_graph, n_node = in_degree.shape
    nt = node_type.reshape(-1).astype(jnp.int32)
    ind = in_degree.reshape(-1).astype(jnp.int32)
    outd = out_degree.reshape(-1).astype(jnp.int32)
    out = _run(nt, ind, outd, node_table, in_degree_table, out_degree_table)
    return out.reshape(n_graph, n_node, EMBED)


# speedup vs baseline: 3.0213x; 1.5910x over previous
import jax
import jax.numpy as jnp
from jax import lax
from jax.experimental import pallas as pl
from jax.experimental.pallas import tpu as pltpu
from jax.experimental.pallas import tpu_sc as plsc

NC = 2
NS = 16
NW = NC * NS
L = 16

EMBED = 768
R_TOTAL = 1024 * 128
ROWS_PER_W = R_TOTAL // NW
CHUNK = 16
N_CHUNKS = ROWS_PER_W // CHUNK
VREGS_PER_ROW = EMBED // L


def _sc_kernel(nt_hbm, in_hbm, ot_hbm, node_tab, in_tab, out_tab, out_hbm,
               idx_n, idx_i, idx_o,
               bn0, bi0, bo0, bn1, bi1, bo1, sem_g0, sem_g1, sem_w):
    wid = lax.axis_index("s") * NC + lax.axis_index("c")
    w_base = wid * ROWS_PER_W

    bn = (bn0, bn1)
    bi = (bi0, bi1)
    bo = (bo0, bo1)
    sem_g = (sem_g0, sem_g1)

    pltpu.sync_copy(nt_hbm.at[pl.ds(w_base, ROWS_PER_W)], idx_n)
    pltpu.sync_copy(in_hbm.at[pl.ds(w_base, ROWS_PER_W)], idx_i)
    pltpu.sync_copy(ot_hbm.at[pl.ds(w_base, ROWS_PER_W)], idx_o)

    def gather_descs(c, b):
        s = pl.ds(c * CHUNK, CHUNK)
        return (
            pltpu.make_async_copy(node_tab.at[idx_n.at[s]], bn[b], sem_g[b]),
            pltpu.make_async_copy(in_tab.at[idx_i.at[s]], bi[b], sem_g[b]),
            pltpu.make_async_copy(out_tab.at[idx_o.at[s]], bo[b], sem_g[b]),
        )

    def wb_desc(c, b):
        return pltpu.make_async_copy(
            bn[b], out_hbm.at[pl.ds(w_base + c * CHUNK, CHUNK)], sem_w)

    for d in gather_descs(0, 0):
        d.start()

    @pl.loop(0, N_CHUNKS, step=2)
    def _pair(cc):
        for b in range(2):
            c = cc + b
            nb = 1 - b
            if b == 0:
                @pl.when(cc > 0)
                def _():
                    wb_desc(cc - 1, nb).wait()
                for d in gather_descs(c + 1, nb):
                    d.start()
            else:
                wb_desc(c - 1, nb).wait()

                @pl.when(cc < N_CHUNKS - 2)
                def _():
                    for d in gather_descs(c + 1, nb):
                        d.start()
            for d in gather_descs(c, b):
                d.wait()

            @pl.loop(0, CHUNK)
            def _row(r):
                for k in range(VREGS_PER_ROW):
                    sl = pl.ds(k * L, L)
                    bn[b][r, sl] = bn[b][r, sl] + bi[b][r, sl] + bo[b][r, sl]

            wb_desc(c, b).start()

    wb_desc(N_CHUNKS - 1, 1).wait()


@jax.jit
def _run(nt, ind, outd, node_tab, in_tab, out_tab):
    mesh = plsc.VectorSubcoreMesh(
        core_axis_name="c", subcore_axis_name="s", num_cores=NC,
        num_subcores=NS)
    f = pl.kernel(
        _sc_kernel,
        out_type=jax.ShapeDtypeStruct((R_TOTAL, EMBED), jnp.float32),
        mesh=mesh,
        scratch_types=[
            pltpu.VMEM((ROWS_PER_W,), jnp.int32),
            pltpu.VMEM((ROWS_PER_W,), jnp.int32),
            pltpu.VMEM((ROWS_PER_W,), jnp.int32),
            pltpu.VMEM((CHUNK, EMBED), jnp.float32),
            pltpu.VMEM((CHUNK, EMBED), jnp.float32),
            pltpu.VMEM((CHUNK, EMBED), jnp.float32),
            pltpu.VMEM((CHUNK, EMBED), jnp.float32),
            pltpu.VMEM((CHUNK, EMBED), jnp.float32),
            pltpu.VMEM((CHUNK, EMBED), jnp.float32),
            pltpu.SemaphoreType.DMA,
            pltpu.SemaphoreType.DMA,
            pltpu.SemaphoreType.DMA,
        ],
    )
    return f(nt, ind, outd, node_tab, in_tab, out_tab)


def kernel(node_type, in_degree, out_degree, node_table, in_degree_table,
           out_degree_table):
    n_graph, n_node = in_degree.shape
    nt = node_type.reshape(-1).astype(jnp.int32)
    ind = in_degree.reshape(-1).astype(jnp.int32)
    outd = out_degree.reshape(-1).astype(jnp.int32)
    out = _run(nt, ind, outd, node_table, in_degree_table, out_degree_table)
    return out.reshape(n_graph, n_node, EMBED)
